# baseline (device time: 37769 ns/iter reference)
import jax
import jax.numpy as jnp
from jax import lax
from jax.experimental import pallas as pl
from jax.experimental.pallas import tpu as pltpu

N_DEV = 8
MASKS = (1, 3, 4)
S = 4
SB = 128
CHUNKS = (48, 40, 40)
OFFS = (0, 48, 88)


def kernel(x, Win0, Wout0, Win1, Wout1, Win2, Wout2):
    b, d = x.shape

    def body(*args):
        x_ref = args[0]
        wins = (args[1], args[3], args[5])
        wouts = (args[2], args[4], args[6])
        out_ref = args[7]
        sbufs = [[args[8 + s * 3 + c] for c in range(3)] for s in range(S)]
        rbufs = [[args[8 + 3 * S + s * 3 + c] for c in range(3)] for s in range(S)]
        send_sems = args[8 + 6 * S]
        recv_sems = args[9 + 6 * S]

        my = lax.axis_index("i")

        barrier = pltpu.get_barrier_semaphore()
        for mask in MASKS:
            pl.semaphore_signal(
                barrier, inc=1,
                device_id=(my ^ mask,), device_id_type=pl.DeviceIdType.MESH,
            )
        pl.semaphore_wait(barrier, len(MASKS))

        def gemm(l, xh):
            h = jnp.maximum(
                jnp.dot(xh, wins[l][...], preferred_element_type=jnp.float32),
                0.0,
            )
            a = jnp.dot(h, wouts[l][...], preferred_element_type=jnp.float32)
            return [
                [a[k * SB + OFFS[c]:k * SB + OFFS[c] + CHUNKS[c], :]
                 for c in range(3)]
                for k in range(2)
            ]

        def make_rdma(l, s, r, c):
            return pltpu.make_async_remote_copy(
                src_ref=sbufs[s][c].at[l, r],
                dst_ref=rbufs[s][c].at[l, r],
                send_sem=send_sems.at[l, r, s, c],
                recv_sem=recv_sems.at[l, r, s, c],
                device_id=(my ^ MASKS[(r + c) % 3],),
                device_id_type=pl.DeviceIdType.MESH,
            )

        def issue(l, s, r, chunks):
            rdmas = []
            for c in range(3):
                sbufs[s][c][l, r] = chunks[c]
                rdma = make_rdma(l, s, r, c)
                rdma.start()
                rdmas.append(rdma)
            return rdmas

        def finish_issue(l, s, r, rdmas, chunks):
            new_rdmas = []
            new_chunks = []
            for c in range(3):
                rdmas[c].wait()
                v = chunks[c] + rbufs[s][c][l, r]
                sbufs[s][c][l, r + 1] = v
                rdma = make_rdma(l, s, r + 1, c)
                rdma.start()
                new_rdmas.append(rdma)
                new_chunks.append(v)
            return new_rdmas, new_chunks

        def finish_last(l, s, rdmas, chunks):
            out = []
            for c in range(3):
                rdmas[c].wait()
                out.append(chunks[c] + rbufs[s][c][l, 2])
            return out

        ch = [None] * S
        rd = [None] * S
        for h in range(S // 2):
            strips = gemm(0, x_ref[h * 2 * SB:(h + 1) * 2 * SB, :])
            for k in range(2):
                s = 2 * h + k
                ch[s] = strips[k]
                rd[s] = issue(0, s, 0, ch[s])

        for l in range(3):
            for r in (0, 1):
                for s in range(S):
                    rd[s], ch[s] = finish_issue(l, s, r, rd[s], ch[s])
            for h in range(S // 2):
                for k in range(2):
                    s = 2 * h + k
                    ch[s] = finish_last(l, s, rd[s], ch[s])
                if l < 2:
                    xh = jnp.concatenate(ch[2 * h] + ch[2 * h + 1], axis=0)
                    strips = gemm(l + 1, xh)
                    for k in range(2):
                        s = 2 * h + k
                        ch[s] = strips[k]
                        rd[s] = issue(l + 1, s, 0, ch[s])

        for s in range(S):
            for c in range(3):
                lo = s * SB + OFFS[c]
                out_ref[lo:lo + CHUNKS[c], :] = ch[s][c]

    return pl.pallas_call(
        body,
        out_shape=jax.ShapeDtypeStruct((b, d), jnp.float32),
        in_specs=[pl.BlockSpec(memory_space=pltpu.VMEM)] * 7,
        out_specs=pl.BlockSpec(memory_space=pltpu.VMEM),
        scratch_shapes=[
            pltpu.VMEM((3, 3, CHUNKS[c], d), jnp.float32)
            for _s in range(S) for c in range(3)
        ] + [
            pltpu.VMEM((3, 3, CHUNKS[c], d), jnp.float32)
            for _s in range(S) for c in range(3)
        ] + [
            pltpu.SemaphoreType.DMA((3, 3, S, 3)),
            pltpu.SemaphoreType.DMA((3, 3, S, 3)),
        ],
        compiler_params=pltpu.CompilerParams(collective_id=0),
    )(x, Win0, Wout0, Win1, Wout1, Win2, Wout2)
